# Initial kernel scaffold; baseline (speedup 1.0000x reference)
import jax
import jax.numpy as jnp
from jax import lax
from jax.experimental import pallas as pl
from jax.experimental.pallas import tpu as pltpu
from jax.experimental.pallas import tpu_sc as plsc

VOCAB = 1000000
EMBED = 32
SEQ = 200
BATCH = 4096

NC = 2
NS = 16
NW = NC * NS

TOTAL = BATCH * SEQ
PER_W = TOTAL // NW
CHUNK = 100
NCHUNK = PER_W // CHUNK


def _sc_embed(token_ids_3d, word_table, pos_table):
    mesh = plsc.VectorSubcoreMesh(core_axis_name="c", subcore_axis_name="s",
                                  num_cores=NC, num_subcores=NS)

    def body(idx_hbm, word_hbm, pos_hbm, out_hbm, idx_v, pos_v, buf):
        wid = lax.axis_index("s") * NC + lax.axis_index("c")
        pltpu.sync_copy(idx_hbm.at[wid], idx_v)
        pltpu.sync_copy(pos_hbm.at[pl.ds(0, SEQ)], pos_v)
        row0 = wid * PER_W

        def chunk_body(j, carry):
            pltpu.sync_copy(word_hbm.at[idx_v.at[j]], buf)
            pbase = (j % 2) * CHUNK
            for r in range(CHUNK):
                for h in range(EMBED // 16):
                    c = pl.ds(h * 16, 16)
                    buf[r, c] = buf[r, c] + pos_v[pbase + r, c]
            pltpu.sync_copy(buf, out_hbm.at[pl.ds(row0 + j * CHUNK, CHUNK)])
            return carry

        lax.fori_loop(0, NCHUNK, chunk_body, 0)

    f = pl.kernel(
        body,
        out_type=jax.ShapeDtypeStruct((TOTAL, EMBED), jnp.float32),
        mesh=mesh,
        scratch_types=[
            pltpu.VMEM((NCHUNK, CHUNK), jnp.int32),
            pltpu.VMEM((SEQ, EMBED), jnp.float32),
            pltpu.VMEM((CHUNK, EMBED), jnp.float32),
        ],
    )
    return f(token_ids_3d, word_table, pos_table)


def kernel(token_ids, word_table, pos_table):
    ids = token_ids.astype(jnp.int32).reshape(NW, NCHUNK, CHUNK)
    out = _sc_embed(ids, word_table, pos_table)
    return out.reshape(BATCH, SEQ, EMBED)


# speedup vs baseline: 2.2884x; 2.2884x over previous
import jax
import jax.numpy as jnp
from jax import lax
from jax.experimental import pallas as pl
from jax.experimental.pallas import tpu as pltpu
from jax.experimental.pallas import tpu_sc as plsc

VOCAB = 1000000
EMBED = 32
SEQ = 200
BATCH = 4096

NC = 2
NS = 16
NW = NC * NS

TOTAL = BATCH * SEQ
PER_W = TOTAL // NW
CHUNK = 80
NCHUNK = PER_W // CHUNK


def _sc_embed(token_ids_3d, word_table, pos_table):
    mesh = plsc.VectorSubcoreMesh(core_axis_name="c", subcore_axis_name="s",
                                  num_cores=NC, num_subcores=NS)

    def body(idx_hbm, word_hbm, pos_hbm, out_hbm, idx_v, pos_v, buf):
        wid = lax.axis_index("s") * NC + lax.axis_index("c")
        pltpu.sync_copy(idx_hbm.at[wid], idx_v)
        pltpu.sync_copy(pos_hbm.at[pl.ds(0, SEQ)], pos_v.at[pl.ds(0, SEQ)])
        pltpu.sync_copy(pos_hbm.at[pl.ds(0, CHUNK)], pos_v.at[pl.ds(SEQ, CHUNK)])
        row0 = wid * PER_W

        def chunk_body(j, carry):
            pltpu.sync_copy(word_hbm.at[idx_v.at[j]], buf)
            pbase = lax.rem(j * CHUNK, SEQ)
            for r in range(CHUNK):
                for h in range(EMBED // 16):
                    c = pl.ds(h * 16, 16)
                    buf[r, c] = buf[r, c] + pos_v[pbase + r, c]
            pltpu.sync_copy(buf, out_hbm.at[pl.ds(row0 + j * CHUNK, CHUNK)])
            return carry

        lax.fori_loop(0, NCHUNK, chunk_body, 0)

    f = pl.kernel(
        body,
        out_type=jax.ShapeDtypeStruct((TOTAL, EMBED), jnp.float32),
        mesh=mesh,
        scratch_types=[
            pltpu.VMEM((NCHUNK, CHUNK), jnp.int32),
            pltpu.VMEM((SEQ + CHUNK, EMBED), jnp.float32),
            pltpu.VMEM((CHUNK, EMBED), jnp.float32),
        ],
        compiler_params=pltpu.CompilerParams(use_tc_tiling_on_sc=False),
    )
    return f(token_ids_3d, word_table, pos_table)


def kernel(token_ids, word_table, pos_table):
    ids = token_ids.astype(jnp.int32).reshape(NW, NCHUNK, CHUNK)
    out = _sc_embed(ids, word_table, pos_table)
    return out.reshape(BATCH, SEQ, EMBED)


# speedup vs baseline: 2.6431x; 1.1550x over previous
import jax
import jax.numpy as jnp
from jax import lax
from jax.experimental import pallas as pl
from jax.experimental.pallas import tpu as pltpu
from jax.experimental.pallas import tpu_sc as plsc

VOCAB = 1000000
EMBED = 32
SEQ = 200
BATCH = 4096

NC = 2
NS = 16
NW = NC * NS

ROWS_W = BATCH // NW
CHUNK = 40
CPS = SEQ // CHUNK
NCHUNK = ROWS_W * CPS
RING = 8
AHEAD = 6


def _sc_embed(token_ids, word_table, pos_table):
    mesh = plsc.VectorSubcoreMesh(core_axis_name="c", subcore_axis_name="s",
                                  num_cores=NC, num_subcores=NS)

    def body(idx_hbm, word_hbm, pos_hbm, out_hbm, idx_v, pos_v, *bufs_and_sems):
        bufs = bufs_and_sems[:RING]
        gsem = bufs_and_sems[RING:2 * RING]
        osem = bufs_and_sems[2 * RING:3 * RING]
        wid = lax.axis_index("s") * NC + lax.axis_index("c")
        brow0 = wid * ROWS_W
        pltpu.sync_copy(idx_hbm.at[pl.ds(brow0, ROWS_W)], idx_v)
        pltpu.sync_copy(pos_hbm.at[pl.ds(0, SEQ)], pos_v)

        def gather(j, slot):
            r = j // CPS
            s0 = (j % CPS) * CHUNK
            idx_slice = idx_v.at[r, pl.ds(s0, CHUNK)]
            return pltpu.make_async_copy(word_hbm.at[idx_slice], bufs[slot],
                                         gsem[slot])

        def outcopy(j, slot):
            r = j // CPS
            s0 = (j % CPS) * CHUNK
            dst = out_hbm.at[brow0 + r].at[pl.ds(s0, CHUNK)]
            return pltpu.make_async_copy(bufs[slot], dst, osem[slot])

        for j in range(AHEAD):
            gather(j, j).start()

        def outer(jg, carry):
            for b in range(RING):
                j = jg * RING + b
                jw = j - (RING - AHEAD)
                if b >= RING - AHEAD:
                    outcopy(jw, (b + AHEAD) % RING).wait()
                else:
                    @pl.when(jg > 0)
                    def _():
                        outcopy(jw, (b + AHEAD) % RING).wait()

                @pl.when(j + AHEAD < NCHUNK)
                def _():
                    gather(j + AHEAD, (b + AHEAD) % RING).start()

                gather(j, b).wait()
                pbase = (j % CPS) * CHUNK
                buf = bufs[b]
                for rr in range(CHUNK):
                    for h in range(EMBED // 16):
                        c = pl.ds(h * 16, 16)
                        buf[rr, c] = buf[rr, c] + pos_v[pbase + rr, c]
                outcopy(j, b).start()
            return carry

        lax.fori_loop(0, NCHUNK // RING, outer, 0)
        for j in range(NCHUNK - (RING - AHEAD), NCHUNK):
            outcopy(j, j % RING).wait()

    f = pl.kernel(
        body,
        out_type=jax.ShapeDtypeStruct((BATCH, SEQ, EMBED), jnp.float32),
        mesh=mesh,
        scratch_types=(
            [pltpu.VMEM((ROWS_W, SEQ), jnp.int32),
             pltpu.VMEM((SEQ, EMBED), jnp.float32)]
            + [pltpu.VMEM((CHUNK, EMBED), jnp.float32) for _ in range(RING)]
            + [pltpu.SemaphoreType.DMA for _ in range(2 * RING)]
        ),
        compiler_params=pltpu.CompilerParams(use_tc_tiling_on_sc=False),
    )
    return f(token_ids, word_table, pos_table)


def kernel(token_ids, word_table, pos_table):
    return _sc_embed(token_ids.astype(jnp.int32), word_table, pos_table)
